# baseline (speedup 1.0000x reference)
import functools

import jax
import jax.numpy as jnp
from jax.experimental import pallas as pl
from jax.experimental.pallas import tpu as pltpu

_NSTREAM = 4


def _gcn_kernel(x_ref, at_ref, w1t_ref, b1t_ref, b2_ref, *rest,
                ck, f_hid, nf):
    *w2_refs, o_ref, s_ref, r_ref = rest
    k = pl.program_id(0)
    n = at_ref.shape[1]

    @pl.when(k == 0)
    def _init():
        s_ref[...] = jnp.dot(x_ref[...], at_ref[...],
                             preferred_element_type=jnp.float32)
        n_iota = jax.lax.broadcasted_iota(jnp.int32, (n, nf), 0)
        j_node = jax.lax.broadcasted_iota(jnp.int32, (n, nf), 1) // f_hid
        r_ref[...] = jnp.where(j_node == n_iota, w1t_ref[...], 0.0)
        o_ref[...] = jnp.broadcast_to(b2_ref[...], o_ref.shape)

    acc = jnp.zeros_like(o_ref)
    for t in range(_NSTREAM):
        kg = k * _NSTREAM + t
        h = jnp.dot(s_ref[...], r_ref[:, pl.ds(kg * ck, ck)],
                    preferred_element_type=jnp.float32)
        h = jnp.maximum(h + b1t_ref[:, pl.ds(kg * ck, ck)], 0.0)
        acc = acc + jnp.dot(h, w2_refs[t][...],
                            preferred_element_type=jnp.float32)
    o_ref[...] += acc


@jax.jit
def kernel(a_norm, x, w1, b1, w2t, b2):
    B, N, f_in = x.shape
    f_hid = w1.shape[1]
    y_dim = w2t.shape[1]
    nf = N * f_hid

    a_t = a_norm.T
    w1_t = jnp.tile(w1, (1, N))
    b1_t = jnp.tile(b1, (1, N))
    x_rows = x[..., 0]

    ck = 512
    nk = nf // (_NSTREAM * ck)

    w2_specs = [
        pl.BlockSpec((ck, y_dim),
                     functools.partial(lambda t, k: (_NSTREAM * k + t, 0), t))
        for t in range(_NSTREAM)
    ]

    out = pl.pallas_call(
        functools.partial(_gcn_kernel, ck=ck, f_hid=f_hid, nf=nf),
        out_shape=jax.ShapeDtypeStruct((B, y_dim), jnp.float32),
        grid=(nk,),
        in_specs=[
            pl.BlockSpec((B, N), lambda k: (0, 0)),
            pl.BlockSpec((N, N), lambda k: (0, 0)),
            pl.BlockSpec((1, nf), lambda k: (0, 0)),
            pl.BlockSpec((1, nf), lambda k: (0, 0)),
            pl.BlockSpec((1, y_dim), lambda k: (0, 0)),
        ] + w2_specs,
        out_specs=pl.BlockSpec((B, y_dim), lambda k: (0, 0)),
        scratch_shapes=[pltpu.VMEM((B, N), jnp.float32),
                        pltpu.VMEM((N, nf), jnp.float32)],
        compiler_params=pltpu.CompilerParams(
            dimension_semantics=("arbitrary",),
        ),
    )(x_rows, a_t, w1_t, b1_t, b2, *([w2t] * _NSTREAM))

    return out


# speedup vs baseline: 1.0891x; 1.0891x over previous
import functools

import jax
import jax.numpy as jnp
from jax.experimental import pallas as pl
from jax.experimental.pallas import tpu as pltpu

_CK = 1024
_NBUF = 4


def _gcn_kernel(x_ref, at_ref, w1t_ref, b1t_ref, b2_ref, w2_hbm, o_ref,
                s_ref, r_ref, bufs, sems, *, nk, f_hid, nf):
    n = at_ref.shape[1]

    def start_copy(kc, slot):
        pltpu.make_async_copy(w2_hbm.at[pl.ds(kc * _CK, _CK), :],
                              bufs.at[slot], sems.at[slot]).start()

    for i in range(_NBUF):
        start_copy(i, i)

    s_ref[...] = jnp.dot(x_ref[...], at_ref[...],
                         preferred_element_type=jnp.float32)
    n_iota = jax.lax.broadcasted_iota(jnp.int32, (n, nf), 0)
    j_node = jax.lax.broadcasted_iota(jnp.int32, (n, nf), 1) // f_hid
    r_ref[...] = jnp.where(j_node == n_iota, w1t_ref[...], 0.0)
    o_ref[...] = jnp.broadcast_to(b2_ref[...], o_ref.shape)

    for k in range(nk):
        slot = k % _NBUF
        pltpu.make_async_copy(w2_hbm.at[pl.ds(k * _CK, _CK), :],
                              bufs.at[slot], sems.at[slot]).wait()
        h = jnp.dot(s_ref[...], r_ref[:, k * _CK:(k + 1) * _CK],
                    preferred_element_type=jnp.float32)
        h = jnp.maximum(h + b1t_ref[:, k * _CK:(k + 1) * _CK], 0.0)
        o_ref[...] += jnp.dot(h, bufs[slot],
                              preferred_element_type=jnp.float32)
        if k + _NBUF < nk:
            start_copy(k + _NBUF, slot)


@jax.jit
def kernel(a_norm, x, w1, b1, w2t, b2):
    B, N, f_in = x.shape
    f_hid = w1.shape[1]
    y_dim = w2t.shape[1]
    nf = N * f_hid
    nk = nf // _CK

    a_t = a_norm.T
    w1_t = jnp.tile(w1, (1, N))
    b1_t = jnp.tile(b1, (1, N))
    x_rows = x[..., 0]

    out = pl.pallas_call(
        functools.partial(_gcn_kernel, nk=nk, f_hid=f_hid, nf=nf),
        out_shape=jax.ShapeDtypeStruct((B, y_dim), jnp.float32),
        in_specs=[
            pl.BlockSpec((B, N), lambda: (0, 0)),
            pl.BlockSpec((N, N), lambda: (0, 0)),
            pl.BlockSpec((1, nf), lambda: (0, 0)),
            pl.BlockSpec((1, nf), lambda: (0, 0)),
            pl.BlockSpec((1, y_dim), lambda: (0, 0)),
            pl.BlockSpec(memory_space=pl.ANY),
        ],
        out_specs=pl.BlockSpec((B, y_dim), lambda: (0, 0)),
        scratch_shapes=[
            pltpu.VMEM((B, N), jnp.float32),
            pltpu.VMEM((N, nf), jnp.float32),
            pltpu.VMEM((_NBUF, _CK, y_dim), jnp.float32),
            pltpu.SemaphoreType.DMA((_NBUF,)),
        ],
        compiler_params=pltpu.CompilerParams(
            vmem_limit_bytes=48 * 1024 * 1024,
        ),
    )(x_rows, a_t, w1_t, b1_t, b2, w2t)

    return out
